# Initial kernel scaffold; baseline (speedup 1.0000x reference)
import jax
import jax.numpy as jnp
from jax.experimental import pallas as pl


def kernel(query, key, value, w_gate, w_noise, in_proj_w, in_proj_b, out_proj_w, out_proj_b):
    raise NotImplementedError("write your pallas kernel here")


# speedup vs baseline: 11.6768x; 11.6768x over previous
import math

import jax
import jax.numpy as jnp
import numpy as np
from jax.experimental import pallas as pl
from jax.experimental.pallas import tpu as pltpu

_NUM_EXPERTS = 8
_TOP_K = 2
_EMBED = 768
_HEADS = 12
_SEQ = 256
_BATCH = 16
_HEAD_DIM = _EMBED // _HEADS
_EPS = float(np.finfo(np.float64).eps)


def _gating_body(q_ref, wg_ref, wn_ref, noise_ref, idx_ref, loss_ref):
    B, N = _BATCH, _NUM_EXPERTS
    qsum = jnp.sum(q_ref[...], axis=0)
    clean = jnp.dot(qsum, wg_ref[...], preferred_element_type=jnp.float32)
    rawn = jnp.dot(qsum, wn_ref[...], preferred_element_type=jnp.float32)
    std = jax.nn.softplus(rawn) + 0.01
    noisy = clean + noise_ref[...] * std
    iota = jax.lax.broadcasted_iota(jnp.int32, (B, N), 1)
    masked = noisy
    vals, idxs = [], []
    for _ in range(_TOP_K + 1):
        v = jnp.max(masked, axis=1, keepdims=True)
        sel = jnp.min(jnp.where(masked >= v, iota, N), axis=1, keepdims=True)
        vals.append(v)
        idxs.append(sel)
        masked = jnp.where(iota == sel, -jnp.inf, masked)
    v0, v1, v2 = vals
    i0, i1 = idxs[0], idxs[1]
    e1 = jnp.exp(v1 - v0)
    g0 = 1.0 / (1.0 + e1)
    g1 = e1 / (1.0 + e1)
    gates = jnp.where(iota == i0, g0, 0.0) + jnp.where(iota == i1, g1, 0.0)
    importance = jnp.sum(gates, axis=0, keepdims=True)
    inv_sqrt2 = 0.7071067811865476
    is_in = noisy > v2
    prob_in = 0.5 * (1.0 + jax.lax.erf((clean - v2) / std * inv_sqrt2))
    prob_out = 0.5 * (1.0 + jax.lax.erf((clean - v1) / std * inv_sqrt2))
    load = jnp.sum(jnp.where(is_in, prob_in, prob_out), axis=0, keepdims=True)

    def cv2(x):
        mu = jnp.mean(x)
        var = jnp.sum((x - mu) ** 2) / (N - 1)
        return var / (mu * mu + 1e-10)

    loss_ref[...] = ((cv2(importance) + cv2(load)) * 0.01).reshape(1, 1)
    idx_ref[...] = jnp.concatenate([i0, i1], axis=1).astype(jnp.int32)


def _moe_body(idx_ref, q_ref, k_ref, v_ref, wi_ref, bi_ref, wo_ref, bo_ref,
              out_ref, w_ref):
    E, H, dh = _EMBED, _HEADS, _HEAD_DIM
    kstep = pl.program_id(1)
    q = q_ref[0]
    kk = k_ref[0]
    vv = v_ref[0]
    Wi = wi_ref[0]
    bi = bi_ref[0]
    Wo = wo_ref[0]
    bo = bo_ref[0]

    def nt(a, b):
        return jax.lax.dot_general(a, b, (((1,), (1,)), ((), ())),
                                   preferred_element_type=jnp.float32)

    qp = nt(q, Wi[:E]) + bi[:, :E]
    kp = nt(kk, Wi[E:2 * E]) + bi[:, E:2 * E]
    vp = nt(vv, Wi[2 * E:]) + bi[:, 2 * E:]
    scale = 1.0 / math.sqrt(dh)
    attn_sum = jnp.zeros((_SEQ, _SEQ), jnp.float32)
    parts = []
    for h in range(H):
        sl = slice(h * dh, (h + 1) * dh)
        attn = jax.nn.softmax(nt(qp[:, sl], kp[:, sl]) * scale, axis=-1)
        attn_sum = attn_sum + attn
        parts.append(jnp.dot(attn, vp[:, sl], preferred_element_type=jnp.float32))
    out = nt(jnp.concatenate(parts, axis=1), Wo) + bo
    exp_out = jnp.exp(out)
    exp_w = jnp.exp(attn_sum * (1.0 / H))

    @pl.when(kstep == 0)
    def _():
        out_ref[0] = exp_out
        w_ref[0] = exp_w

    @pl.when(kstep == _TOP_K - 1)
    def _():
        tot = out_ref[0] + exp_out
        out_ref[0] = jnp.log(jnp.where(tot == 0.0, _EPS, tot))
        totw = w_ref[0] + exp_w
        w_ref[0] = jnp.log(jnp.where(totw == 0.0, _EPS, totw))


def kernel(query, key, value, w_gate, w_noise, in_proj_w, in_proj_b,
           out_proj_w, out_proj_b):
    S, B, E = _SEQ, _BATCH, _EMBED
    noise = jax.random.normal(jax.random.key(1234), (B, _NUM_EXPERTS),
                              dtype=jnp.float32)
    idx, loss2 = pl.pallas_call(
        _gating_body,
        out_shape=(
            jax.ShapeDtypeStruct((B, _TOP_K), jnp.int32),
            jax.ShapeDtypeStruct((1, 1), jnp.float32),
        ),
    )(query, w_gate, w_noise, noise)
    loss = loss2[0, 0]

    bi3 = in_proj_b.reshape(_NUM_EXPERTS, 1, 3 * E)
    bo3 = out_proj_b.reshape(_NUM_EXPERTS, 1, E)
    q_bse = jnp.transpose(query, (1, 0, 2))
    k_bse = jnp.transpose(key, (1, 0, 2))
    v_bse = jnp.transpose(value, (1, 0, 2))

    grid_spec = pltpu.PrefetchScalarGridSpec(
        num_scalar_prefetch=1,
        grid=(B, _TOP_K),
        in_specs=[
            pl.BlockSpec((1, S, E), lambda b, k, idx: (b, 0, 0)),
            pl.BlockSpec((1, S, E), lambda b, k, idx: (b, 0, 0)),
            pl.BlockSpec((1, S, E), lambda b, k, idx: (b, 0, 0)),
            pl.BlockSpec((1, 3 * E, E), lambda b, k, idx: (idx[b, k], 0, 0)),
            pl.BlockSpec((1, 1, 3 * E), lambda b, k, idx: (idx[b, k], 0, 0)),
            pl.BlockSpec((1, E, E), lambda b, k, idx: (idx[b, k], 0, 0)),
            pl.BlockSpec((1, 1, E), lambda b, k, idx: (idx[b, k], 0, 0)),
        ],
        out_specs=[
            pl.BlockSpec((1, S, E), lambda b, k, idx: (b, 0, 0)),
            pl.BlockSpec((1, S, S), lambda b, k, idx: (b, 0, 0)),
        ],
    )
    out_bse, w_log = pl.pallas_call(
        _moe_body,
        grid_spec=grid_spec,
        out_shape=(
            jax.ShapeDtypeStruct((B, S, E), jnp.float32),
            jax.ShapeDtypeStruct((B, S, S), jnp.float32),
        ),
    )(idx, q_bse, k_bse, v_bse, in_proj_w, bi3, out_proj_w, bo3)
    return (jnp.transpose(out_bse, (1, 0, 2)), loss, w_log)
